# baseline (device time: 39450 ns/iter reference)
import jax
import jax.numpy as jnp
from jax import lax
from jax.experimental import pallas as pl
from jax.experimental.pallas import tpu as pltpu

N_DEV = 4
SQ = 512
D = 1024
H = 8
DH = 128
SCALE = 0.08838834764831843
NC = 4
RC = SQ // NC
HW = D // 2
LAG = 2


def _body(x_ref, wq_ref, wo_ref, wk_ref, wv_ref, out_ref,
          x_s, w_s, out_s, send_ref, recv_ref,
          load_sems, store_sems, send_sems, recv_sems):
    my = lax.axis_index("i")
    p1 = my ^ 1
    p2 = 3 - my
    partners = [(p1, p2), (p2, p1)]

    loads = [
        pltpu.make_async_copy(x_ref.at[0], x_s, load_sems.at[0]),
        pltpu.make_async_copy(wk_ref, w_s.at[0], load_sems.at[1]),
        pltpu.make_async_copy(wv_ref, w_s.at[1], load_sems.at[2]),
        pltpu.make_async_copy(wq_ref, w_s.at[2], load_sems.at[3]),
        pltpu.make_async_copy(wo_ref, w_s.at[3], load_sems.at[4]),
    ]
    for ld in loads:
        ld.start()

    dn = (((1,), (0,)), ((), ()))
    loads[0].wait()
    xv = x_s[...].astype(jnp.bfloat16)
    loads[1].wait()
    k_all = lax.dot_general(xv, w_s[0].astype(jnp.bfloat16), dn,
                            preferred_element_type=jnp.float32).astype(jnp.bfloat16)
    loads[2].wait()
    v_all = lax.dot_general(xv, w_s[1].astype(jnp.bfloat16), dn,
                            preferred_element_type=jnp.float32).astype(jnp.bfloat16)
    loads[3].wait()
    xq = xv * jnp.bfloat16(SCALE)
    q_all = lax.dot_general(xq, w_s[2].astype(jnp.bfloat16), dn,
                            preferred_element_type=jnp.float32).astype(jnp.bfloat16)

    outs = []
    for h in range(H):
        sl = slice(h * DH, (h + 1) * DH)
        s = lax.dot_general(q_all[:, sl], k_all[:, sl],
                            (((1,), (1,)), ((), ())),
                            preferred_element_type=jnp.float32)
        p = jnp.exp(s)
        l = jnp.sum(p, axis=1, keepdims=True)
        o = lax.dot_general(p.astype(jnp.bfloat16), v_all[:, sl], dn,
                            preferred_element_type=jnp.float32)
        outs.append((o / l).astype(jnp.bfloat16))
    attn = jnp.concatenate(outs, axis=1)

    loads[4].wait()
    wo = w_s[3].astype(jnp.bfloat16)

    def make_rdma(c, stage, half):
        return pltpu.make_async_remote_copy(
            src_ref=send_ref.at[c, stage, half],
            dst_ref=recv_ref.at[c, stage, half],
            send_sem=send_sems.at[c, stage, half],
            recv_sem=recv_sems.at[c, stage, half],
            device_id=(partners[stage][half],),
            device_id_type=pl.DeviceIdType.MESH,
        )

    def start_stage1(c):
        partial = lax.dot_general(attn[c * RC:(c + 1) * RC, :], wo, dn,
                                  preferred_element_type=jnp.float32
                                  ).astype(jnp.bfloat16)
        rdmas = []
        for half in range(2):
            send_ref[c, 0, half] = partial[:, half * HW:(half + 1) * HW]
            r = make_rdma(c, 0, half)
            r.start()
            rdmas.append(r)
        return rdmas, [partial[:, :HW].astype(jnp.float32),
                       partial[:, HW:].astype(jnp.float32)]

    def advance_stage2(c, s1_state):
        rdmas, halves = s1_state
        summed = []
        r2 = []
        for half in range(2):
            rdmas[half].wait()
            v = halves[half] + recv_ref[c, 0, half].astype(jnp.float32)
            send_ref[c, 1, half] = v.astype(jnp.bfloat16)
            r = make_rdma(c, 1, half)
            r.start()
            r2.append(r)
            summed.append(v)
        return r2, summed

    stores = []

    def finish(c, s2_state):
        rdmas, halves = s2_state
        for half in range(2):
            rdmas[half].wait()
            out_s[c * RC:(c + 1) * RC, half * HW:(half + 1) * HW] = (
                halves[half] + recv_ref[c, 1, half].astype(jnp.float32))
        st = pltpu.make_async_copy(
            out_s.at[c * RC:(c + 1) * RC, :],
            out_ref.at[c * RC:(c + 1) * RC, :],
            store_sems.at[c],
        )
        st.start()
        stores.append(st)

    s1 = [None] * NC
    s2 = [None] * NC
    for c in range(NC):
        s1[c] = start_stage1(c)
        if c >= LAG:
            s2[c - LAG] = advance_stage2(c - LAG, s1[c - LAG])
        if c >= 2 * LAG:
            finish(c - 2 * LAG, s2[c - 2 * LAG])
    for c in range(NC - LAG, NC):
        s2[c] = advance_stage2(c, s1[c])
    for c in range(max(NC - 2 * LAG, 0), NC):
        finish(c, s2[c])
    for st in stores:
        st.wait()


def kernel(x, Wq, Wo, Wk, Wv):
    out = pl.pallas_call(
        _body,
        out_shape=jax.ShapeDtypeStruct((SQ, D), jnp.float32),
        in_specs=[pl.BlockSpec(memory_space=pltpu.MemorySpace.HBM)] * 5,
        out_specs=pl.BlockSpec(memory_space=pltpu.MemorySpace.HBM),
        scratch_shapes=[
            pltpu.VMEM((SQ, D), jnp.float32),
            pltpu.VMEM((4, D, D), jnp.float32),
            pltpu.VMEM((SQ, D), jnp.float32),
            pltpu.VMEM((NC, 2, 2, RC, HW), jnp.bfloat16),
            pltpu.VMEM((NC, 2, 2, RC, HW), jnp.bfloat16),
            pltpu.SemaphoreType.DMA((5,)),
            pltpu.SemaphoreType.DMA((NC,)),
            pltpu.SemaphoreType.DMA((NC, 2, 2)),
            pltpu.SemaphoreType.DMA((NC, 2, 2)),
        ],
    )(x, Wq, Wo, Wk, Wv)
    return out.reshape(1, SQ, D)


# device time: 34925 ns/iter; 1.1296x vs baseline; 1.1296x over previous
import jax
import jax.numpy as jnp
from jax import lax
from jax.experimental import pallas as pl
from jax.experimental.pallas import tpu as pltpu

N_DEV = 4
SQ = 512
D = 1024
H = 8
DH = 128
SCALE = 0.08838834764831843
NC = 4
RC = SQ // NC
HW = D // 2
LAG = 2


def _body(x_ref, wq_ref, wo_ref, wk_ref, wv_ref, out_ref,
          send_ref, recv_ref, send_sems, recv_sems):
    my = lax.axis_index("i")
    p1 = my ^ 1
    p2 = 3 - my
    partners = [(p1, p2), (p2, p1)]

    barrier_sem = pltpu.get_barrier_semaphore()
    for nbr in (p1, p2):
        pl.semaphore_signal(barrier_sem, inc=1, device_id=(nbr,),
                            device_id_type=pl.DeviceIdType.MESH)
    pl.semaphore_wait(barrier_sem, 2)

    xv = x_ref[...].astype(jnp.bfloat16)
    wq = wq_ref[...].astype(jnp.bfloat16)
    wk = wk_ref[...].astype(jnp.bfloat16)
    wv = wv_ref[...].astype(jnp.bfloat16)
    wo = wo_ref[...].astype(jnp.bfloat16)
    xq = xv * jnp.bfloat16(SCALE)

    dn = (((1,), (0,)), ((), ()))
    q_all = lax.dot_general(xq, wq, dn,
                            preferred_element_type=jnp.float32).astype(jnp.bfloat16)
    k_all = lax.dot_general(xv, wk, dn,
                            preferred_element_type=jnp.float32).astype(jnp.bfloat16)
    v_all = lax.dot_general(xv, wv, dn,
                            preferred_element_type=jnp.float32).astype(jnp.bfloat16)

    outs = []
    for h in range(H):
        sl = slice(h * DH, (h + 1) * DH)
        s = lax.dot_general(q_all[:, sl], k_all[:, sl],
                            (((1,), (1,)), ((), ())),
                            preferred_element_type=jnp.float32)
        p = jnp.exp(s)
        l = jnp.sum(p, axis=1, keepdims=True)
        o = lax.dot_general(p.astype(jnp.bfloat16), v_all[:, sl], dn,
                            preferred_element_type=jnp.float32)
        outs.append((o / l).astype(jnp.bfloat16))
    attn = jnp.concatenate(outs, axis=1)

    def make_rdma(c, stage, half):
        return pltpu.make_async_remote_copy(
            src_ref=send_ref.at[c, stage, half],
            dst_ref=recv_ref.at[c, stage, half],
            send_sem=send_sems.at[c, stage, half],
            recv_sem=recv_sems.at[c, stage, half],
            device_id=(partners[stage][half],),
            device_id_type=pl.DeviceIdType.MESH,
        )

    def start_stage1(c):
        partial = lax.dot_general(attn[c * RC:(c + 1) * RC, :], wo, dn,
                                  preferred_element_type=jnp.float32
                                  ).astype(jnp.bfloat16)
        rdmas = []
        for half in range(2):
            send_ref[c, 0, half] = partial[:, half * HW:(half + 1) * HW]
            r = make_rdma(c, 0, half)
            r.start()
            rdmas.append(r)
        return rdmas, [partial[:, :HW].astype(jnp.float32),
                       partial[:, HW:].astype(jnp.float32)]

    def advance_stage2(c, s1_state):
        rdmas, halves = s1_state
        summed = []
        r2 = []
        for half in range(2):
            rdmas[half].wait_recv()
            all_rdmas.append(rdmas[half])
            v = halves[half] + recv_ref[c, 0, half].astype(jnp.float32)
            send_ref[c, 1, half] = v.astype(jnp.bfloat16)
            r = make_rdma(c, 1, half)
            r.start()
            r2.append(r)
            summed.append(v)
        return r2, summed

    def finish(c, s2_state):
        rdmas, halves = s2_state
        for half in range(2):
            rdmas[half].wait_recv()
            all_rdmas.append(rdmas[half])
            out_ref[c * RC:(c + 1) * RC, half * HW:(half + 1) * HW] = (
                halves[half] + recv_ref[c, 1, half].astype(jnp.float32))

    all_rdmas = []
    s1 = [None] * NC
    s2 = [None] * NC
    for c in range(NC):
        s1[c] = start_stage1(c)
        if c >= LAG:
            s2[c - LAG] = advance_stage2(c - LAG, s1[c - LAG])
        if c >= 2 * LAG:
            finish(c - 2 * LAG, s2[c - 2 * LAG])
    for c in range(NC - LAG, NC):
        s2[c] = advance_stage2(c, s1[c])
    for c in range(max(NC - 2 * LAG, 0), NC):
        finish(c, s2[c])
    for r in all_rdmas:
        r.wait_send()


def kernel(x, Wq, Wo, Wk, Wv):
    x2 = x.reshape(SQ, D)
    out = pl.pallas_call(
        _body,
        out_shape=jax.ShapeDtypeStruct((SQ, D), jnp.float32),
        in_specs=[pl.BlockSpec(memory_space=pltpu.VMEM)] * 5,
        out_specs=pl.BlockSpec(memory_space=pltpu.VMEM),
        scratch_shapes=[
            pltpu.VMEM((NC, 2, 2, RC, HW), jnp.bfloat16),
            pltpu.VMEM((NC, 2, 2, RC, HW), jnp.bfloat16),
            pltpu.SemaphoreType.DMA((NC, 2, 2)),
            pltpu.SemaphoreType.DMA((NC, 2, 2)),
        ],
        compiler_params=pltpu.CompilerParams(collective_id=0),
    )(x2, Wq, Wo, Wk, Wv)
    return out.reshape(1, SQ, D)
